# baseline (speedup 1.0000x reference)
import functools

import jax
import jax.numpy as jnp
from jax import lax
from jax.experimental import pallas as pl
from jax.experimental.pallas import tpu as pltpu
from jax.experimental.pallas import tpu_sc as plsc
from jax._src.pallas import mpmd as _mpmd

NC = 2
NS = 16
L = 16
CH = 128
W = 128


def _sc_mesh():
    return plsc.VectorSubcoreMesh(core_axis_name="c", subcore_axis_name="s",
                                  num_cores=NC, num_subcores=NS)


def _sc_params():
    return pltpu.CompilerParams(use_tc_tiling_on_sc=False)


def _make_to_rows(M, D, lb):
    P = W // D
    R = lb // P

    def body(x, o):
        t = x[...].T
        t4 = t.reshape(R, P, D)
        o[...] = jnp.concatenate([t4[:, j, :] for j in range(P)], axis=1)

    return pl.pallas_call(
        body,
        grid=(pl.cdiv(M, lb),),
        in_specs=[pl.BlockSpec((D, lb), lambda i: (0, i))],
        out_specs=pl.BlockSpec((R, W), lambda i: (i, 0)),
        out_shape=jax.ShapeDtypeStruct((M // P, W), jnp.float32),
    )


def _make_to_native(M, D, lb):
    P = W // D
    R = lb // P

    def body(x, o):
        xb = x[...]
        t4 = jnp.stack([xb[:, j * D:(j + 1) * D] for j in range(P)], axis=1)
        o[...] = t4.reshape(lb, D).T

    return pl.pallas_call(
        body,
        grid=(pl.cdiv(M, lb),),
        in_specs=[pl.BlockSpec((R, W), lambda i: (i, 0))],
        out_specs=pl.BlockSpec((D, lb), lambda i: (0, i)),
        out_shape=jax.ShapeDtypeStruct((D, M), jnp.float32),
    )


def _make_elect(M, B):
    EV = B // NS
    NCHUNK = EV // CH

    def body(idx2, rep_o, idx2d, rep2d, e1d, T):
        cid = lax.axis_index("c")
        sid = lax.axis_index("s")
        base = sid * EV
        rows_per_tile = EV // CH

        @pl.when(cid == 0)
        def _():
            pltpu.sync_copy(idx2.at[pl.ds(sid * rows_per_tile,
                                          rows_per_tile)], idx2d)

            def init_body(i, carry):
                e1d[pl.ds(i * L, L)] = (
                    jnp.full((L,), base, dtype=jnp.int32)
                    + i * L + lax.iota(jnp.int32, L))
                return carry
            lax.fori_loop(0, EV // L, init_body, 0)
            for j in range(NCHUNK):
                pltpu.sync_copy(e1d.at[pl.ds(j * CH, CH)],
                                T.at[idx2d.at[j]])

        plsc.subcore_barrier()

        @pl.when(cid == 0)
        def _():
            for j in range(NCHUNK):
                pltpu.sync_copy(T.at[idx2d.at[j]], rep2d.at[j])
            pltpu.sync_copy(rep2d,
                            rep_o.at[pl.ds(sid * rows_per_tile,
                                           rows_per_tile)])

    out_type = jax.ShapeDtypeStruct((B // CH, CH), jnp.int32)
    scratch = [
        pltpu.VMEM((EV // CH, CH), jnp.int32),
        pltpu.VMEM((EV // CH, CH), jnp.int32),
        pltpu.VMEM((EV,), jnp.int32),
        pltpu.VMEM_SHARED((M,), jnp.int32),
    ]
    return pl.kernel(body, out_type=out_type, mesh=_sc_mesh(),
                     scratch_types=scratch, compiler_params=_sc_params())


def _make_hgather(M, B, D):
    NW = NC * NS
    EV = B // NW
    NCHUNK = EV // CH

    def body(tab, idx2, h_o, idx2d, rows_v):
        cid = lax.axis_index("c")
        sid = lax.axis_index("s")
        w = cid * NS + sid
        rows_per_w = EV // CH
        pltpu.sync_copy(idx2.at[pl.ds(w * rows_per_w, rows_per_w)], idx2d)
        for j in range(NCHUNK):
            pltpu.sync_copy(tab.at[idx2d.at[j]],
                            rows_v.at[pl.ds(j * CH, CH)])
        pltpu.sync_copy(rows_v, h_o.at[pl.ds(w * EV, EV)])

    out_type = jax.ShapeDtypeStruct((B, D), jnp.float32)
    scratch = [
        pltpu.VMEM((EV // CH, CH), jnp.int32),
        pltpu.VMEM((EV, D), jnp.float32),
    ]
    return pl.kernel(body, out_type=out_type, mesh=_sc_mesh(),
                     scratch_types=scratch, compiler_params=_sc_params())


def _make_aggregate(B, D):
    EV = B // NS
    NCHUNK = EV // CH

    def body(rep2, val, S_o, c_o,
             rep2d, ones1d, z1v, val_v, rows_v, acc, cnt):
        cid = lax.axis_index("c")
        sid = lax.axis_index("s")
        base = sid * EV
        rows_per_tile = EV // CH

        @pl.when(cid == 0)
        def _():
            pltpu.sync_copy(rep2.at[pl.ds(sid * rows_per_tile,
                                          rows_per_tile)], rep2d)
            pltpu.sync_copy(val.at[pl.ds(base, EV)], val_v)

            def init_body(i, carry):
                ones1d[pl.ds(i * L, L)] = jnp.full((L,), 1.0,
                                                   dtype=jnp.float32)
                z1v[pl.ds(i * L, L)] = jnp.full((L,), 0.0, dtype=jnp.float32)
                return carry
            lax.fori_loop(0, EV // L, init_body, 0)

            def zero_body(i, carry):
                rows_v[i, pl.ds(0, L)] = jnp.full((L,), 0.0,
                                                  dtype=jnp.float32)
                rows_v[i, pl.ds(L, L)] = jnp.full((L,), 0.0,
                                                  dtype=jnp.float32)
                return carry
            lax.fori_loop(0, EV, zero_body, 0)

            pltpu.sync_copy(rows_v, acc.at[pl.ds(base, EV)])
            pltpu.sync_copy(z1v, cnt.at[pl.ds(base, EV)])

        plsc.subcore_barrier()

        @pl.when(cid == 0)
        def _():
            for j in range(NCHUNK):
                pltpu.sync_copy(val_v.at[pl.ds(j * CH, CH)],
                                acc.at[rep2d.at[j]], add=True)
                pltpu.sync_copy(ones1d.at[pl.ds(j * CH, CH)],
                                cnt.at[rep2d.at[j]], add=True)

        plsc.subcore_barrier()

        @pl.when(cid == 0)
        def _():
            for j in range(NCHUNK):
                pltpu.sync_copy(acc.at[rep2d.at[j]],
                                rows_v.at[pl.ds(j * CH, CH)])
                pltpu.sync_copy(cnt.at[rep2d.at[j]],
                                ones1d.at[pl.ds(j * CH, CH)])
            pltpu.sync_copy(rows_v, S_o.at[pl.ds(base, EV)])
            pltpu.sync_copy(ones1d, c_o.at[pl.ds(base, EV)])

    out_type = (
        jax.ShapeDtypeStruct((B, D), jnp.float32),
        jax.ShapeDtypeStruct((B,), jnp.float32),
    )
    scratch = [
        pltpu.VMEM((EV // CH, CH), jnp.int32),
        pltpu.VMEM((EV,), jnp.float32),
        pltpu.VMEM((EV,), jnp.float32),
        pltpu.VMEM((EV, D), jnp.float32),
        pltpu.VMEM((EV, D), jnp.float32),
        pltpu.VMEM_SHARED((B, D), jnp.float32),
        pltpu.VMEM_SHARED((B,), jnp.float32),
    ]
    return pl.kernel(body, out_type=out_type, mesh=_sc_mesh(),
                     scratch_types=scratch, compiler_params=_sc_params())


def _make_gru(B, D, block):
    def body(S, c, h, Wr, Wz, Wn, Ur, Uz, Un, br, bz, bn, dr, dz, dn, out):
        m = S[...] / jnp.maximum(c[...], 1.0)
        hh = h[...]
        i_r = jnp.dot(m, Wr[...], preferred_element_type=jnp.float32) + br[...]
        i_z = jnp.dot(m, Wz[...], preferred_element_type=jnp.float32) + bz[...]
        i_n = jnp.dot(m, Wn[...], preferred_element_type=jnp.float32) + bn[...]
        h_r = jnp.dot(hh, Ur[...], preferred_element_type=jnp.float32) + dr[...]
        h_z = jnp.dot(hh, Uz[...], preferred_element_type=jnp.float32) + dz[...]
        h_n = jnp.dot(hh, Un[...], preferred_element_type=jnp.float32) + dn[...]
        r = jax.nn.sigmoid(i_r + h_r)
        z = jax.nn.sigmoid(i_z + h_z)
        n = jnp.tanh(i_n + r * h_n)
        out[:, pl.ds(0, D)] = ((1.0 - z) * n + z * hh)

    grid = (B // block,)
    row_spec = pl.BlockSpec((block, D), lambda i: (i, 0))
    c_spec = pl.BlockSpec((block, 1), lambda i: (i, 0))
    w_spec = pl.BlockSpec((D, D), lambda i: (0, 0))
    b_spec = pl.BlockSpec((1, D), lambda i: (0, 0))
    return pl.pallas_call(
        body,
        grid=grid,
        in_specs=[row_spec, c_spec, row_spec] + [w_spec] * 6 + [b_spec] * 6,
        out_specs=row_spec,
        out_shape=jax.ShapeDtypeStruct((B, D), jnp.float32),
    )


def _make_scatter(M, B, D):
    NW = NC * NS
    EV = B // NW
    NCHUNK = EV // CH

    def body(tab_in, idx2, hnew, out, idx2d, h_v):
        cid = lax.axis_index("c")
        sid = lax.axis_index("s")
        w = cid * NS + sid
        rows_per_w = EV // CH
        pltpu.sync_copy(idx2.at[pl.ds(w * rows_per_w, rows_per_w)], idx2d)
        pltpu.sync_copy(hnew.at[pl.ds(w * EV, EV)], h_v)
        for j in range(NCHUNK):
            pltpu.sync_copy(h_v.at[pl.ds(j * CH, CH)], out.at[idx2d.at[j]])

    scratch = [
        pltpu.VMEM((EV // CH, CH), jnp.int32),
        pltpu.VMEM((EV, D), jnp.float32),
    ]
    return _mpmd._mpmd_map(
        [(_sc_mesh(), body)],
        out_types=jax.ShapeDtypeStruct((M, D), jnp.float32),
        input_output_aliases={0: 0},
        scratch_types=scratch,
        compiler_params=_sc_params(),
    )


def kernel(mem, idx, val, W_ih, W_hh, b_ih, b_hh):
    M, D = mem.shape
    B = idx.shape[0]

    idx2 = idx.astype(jnp.int32).reshape(B // CH, CH)

    mem_t = jnp.swapaxes(mem, 0, 1)
    tab = _make_to_rows(M, D, 8192)(mem_t)
    tab1m = tab.reshape(M, D)

    rep2 = _make_elect(M, B)(idx2)
    S, c = _make_aggregate(B, D)(rep2, val)
    h = _make_hgather(M, B, D)(tab1m, idx2)

    Wr, Wz, Wn = W_ih[:, :D], W_ih[:, D:2 * D], W_ih[:, 2 * D:]
    Ur, Uz, Un = W_hh[:, :D], W_hh[:, D:2 * D], W_hh[:, 2 * D:]
    br, bz, bn = (b_ih[:D].reshape(1, D), b_ih[D:2 * D].reshape(1, D),
                  b_ih[2 * D:].reshape(1, D))
    dr, dz, dn = (b_hh[:D].reshape(1, D), b_hh[D:2 * D].reshape(1, D),
                  b_hh[2 * D:].reshape(1, D))

    h_new = _make_gru(B, D, 2048)(
        S, c.reshape(B, 1), h, Wr, Wz, Wn, Ur, Uz, Un,
        br, bz, bn, dr, dz, dn)

    out_tab = _make_scatter(M, B, D)(tab1m, idx2, h_new)
    out_t = _make_to_native(M, D, 8192)(out_tab.reshape(M * D // W, W))
    return jnp.swapaxes(out_t, 0, 1)


# speedup vs baseline: 2.9447x; 2.9447x over previous
import functools

import jax
import jax.numpy as jnp
from jax import lax
from jax.experimental import pallas as pl
from jax.experimental.pallas import tpu as pltpu
from jax.experimental.pallas import tpu_sc as plsc
from jax._src.pallas import mpmd as _mpmd

NC = 2
NS = 16
L = 16
CH = 128
W = 128


def _sc_mesh():
    return plsc.VectorSubcoreMesh(core_axis_name="c", subcore_axis_name="s",
                                  num_cores=NC, num_subcores=NS)


def _sc_params():
    return pltpu.CompilerParams(use_tc_tiling_on_sc=False)


def _make_to_rows(M, D, lb):
    P = W // D
    R = lb // P
    G = pl.cdiv(M, lb)

    def body(x, o):
        eye = (lax.broadcasted_iota(jnp.int32, (D, D), 0)
               == lax.broadcasted_iota(jnp.int32, (D, D), 1)
               ).astype(jnp.float32)
        for j in range(P):
            o[:, pl.ds(j * D, D)] = lax.dot_general(
                x[:, pl.ds(j * R, R)], eye, (((0,), (0,)), ((), ())),
                preferred_element_type=jnp.float32)

    return pl.pallas_call(
        body,
        grid=(G,),
        in_specs=[pl.BlockSpec((D, lb), lambda i: (0, i))],
        out_specs=pl.BlockSpec((R, W), lambda i: (i, 0)),
        out_shape=jax.ShapeDtypeStruct((G * R, W), jnp.float32),
    )


def _make_to_native(M, D, lb):
    P = W // D
    R = lb // P
    G = pl.cdiv(M, lb)

    def body(x, o):
        eye = (lax.broadcasted_iota(jnp.int32, (D, D), 0)
               == lax.broadcasted_iota(jnp.int32, (D, D), 1)
               ).astype(jnp.float32)
        for j in range(P):
            o[:, pl.ds(j * R, R)] = lax.dot_general(
                eye, x[:, pl.ds(j * D, D)], (((1,), (1,)), ((), ())),
                preferred_element_type=jnp.float32)

    return pl.pallas_call(
        body,
        grid=(G,),
        in_specs=[pl.BlockSpec((R, W), lambda i: (i, 0))],
        out_specs=pl.BlockSpec((D, lb), lambda i: (0, i)),
        out_shape=jax.ShapeDtypeStruct((D, M), jnp.float32),
    )


def _make_elect(M, B):
    EV = B // NS
    NCHUNK = EV // CH

    def body(idx2, rep_o, idx2d, rep2d, e1d, T):
        cid = lax.axis_index("c")
        sid = lax.axis_index("s")
        base = sid * EV
        rows_per_tile = EV // CH

        @pl.when(cid == 0)
        def _():
            pltpu.sync_copy(idx2.at[pl.ds(sid * rows_per_tile,
                                          rows_per_tile)], idx2d)

            def init_body(i, carry):
                e1d[pl.ds(i * L, L)] = (
                    jnp.full((L,), base, dtype=jnp.int32)
                    + i * L + lax.iota(jnp.int32, L))
                return carry
            lax.fori_loop(0, EV // L, init_body, 0)
            for j in range(NCHUNK):
                pltpu.sync_copy(e1d.at[pl.ds(j * CH, CH)],
                                T.at[idx2d.at[j]])

        plsc.subcore_barrier()

        @pl.when(cid == 0)
        def _():
            for j in range(NCHUNK):
                pltpu.sync_copy(T.at[idx2d.at[j]], rep2d.at[j])
            pltpu.sync_copy(rep2d,
                            rep_o.at[pl.ds(sid * rows_per_tile,
                                           rows_per_tile)])

    out_type = jax.ShapeDtypeStruct((B // CH, CH), jnp.int32)
    scratch = [
        pltpu.VMEM((EV // CH, CH), jnp.int32),
        pltpu.VMEM((EV // CH, CH), jnp.int32),
        pltpu.VMEM((EV,), jnp.int32),
        pltpu.VMEM_SHARED((M,), jnp.int32),
    ]
    return pl.kernel(body, out_type=out_type, mesh=_sc_mesh(),
                     scratch_types=scratch, compiler_params=_sc_params())


def _row_permute(idx2d, nrows):
    for r in range(nrows):
        for k in range(CH // L):
            v = idx2d[r, pl.ds(k * L, L)]
            w = ((v & (-8192)) | ((v & 2047) << 2)
                 | (lax.shift_right_logical(v, 11) & 3))
            idx2d[r, pl.ds(k * L, L)] = w


def _make_hgather(M, B, D):
    NW = NC * NS
    EV = B // NW
    NCHUNK = EV // CH

    def body(tab, idx2, h_o, idx2d, rows_v):
        cid = lax.axis_index("c")
        sid = lax.axis_index("s")
        w = cid * NS + sid
        rows_per_w = EV // CH
        pltpu.sync_copy(idx2.at[pl.ds(w * rows_per_w, rows_per_w)], idx2d)
        _row_permute(idx2d, rows_per_w)
        for j in range(NCHUNK):
            pltpu.sync_copy(tab.at[idx2d.at[j]],
                            rows_v.at[pl.ds(j * CH, CH)])
        pltpu.sync_copy(rows_v, h_o.at[pl.ds(w * EV, EV)])

    out_type = jax.ShapeDtypeStruct((B, D), jnp.float32)
    scratch = [
        pltpu.VMEM((EV // CH, CH), jnp.int32),
        pltpu.VMEM((EV, D), jnp.float32),
    ]
    return pl.kernel(body, out_type=out_type, mesh=_sc_mesh(),
                     scratch_types=scratch, compiler_params=_sc_params())


def _make_aggregate(B, D):
    EV = B // NS
    NCHUNK = EV // CH

    def body(rep2, val, S_o, c_o,
             rep2d, ones1d, z1v, val_v, rows_v, acc, cnt):
        cid = lax.axis_index("c")
        sid = lax.axis_index("s")
        base = sid * EV
        rows_per_tile = EV // CH

        @pl.when(cid == 0)
        def _():
            pltpu.sync_copy(rep2.at[pl.ds(sid * rows_per_tile,
                                          rows_per_tile)], rep2d)
            pltpu.sync_copy(val.at[pl.ds(base, EV)], val_v)

            def init_body(i, carry):
                ones1d[pl.ds(i * L, L)] = jnp.full((L,), 1.0,
                                                   dtype=jnp.float32)
                z1v[pl.ds(i * L, L)] = jnp.full((L,), 0.0, dtype=jnp.float32)
                return carry
            lax.fori_loop(0, EV // L, init_body, 0)

            def zero_body(i, carry):
                rows_v[i, pl.ds(0, L)] = jnp.full((L,), 0.0,
                                                  dtype=jnp.float32)
                rows_v[i, pl.ds(L, L)] = jnp.full((L,), 0.0,
                                                  dtype=jnp.float32)
                return carry
            lax.fori_loop(0, EV, zero_body, 0)

            pltpu.sync_copy(rows_v, acc.at[pl.ds(base, EV)])
            pltpu.sync_copy(z1v, cnt.at[pl.ds(base, EV)])

        plsc.subcore_barrier()

        @pl.when(cid == 0)
        def _():
            for j in range(NCHUNK):
                pltpu.sync_copy(val_v.at[pl.ds(j * CH, CH)],
                                acc.at[rep2d.at[j]], add=True)
                pltpu.sync_copy(ones1d.at[pl.ds(j * CH, CH)],
                                cnt.at[rep2d.at[j]], add=True)

        plsc.subcore_barrier()

        @pl.when(cid == 0)
        def _():
            for j in range(NCHUNK):
                pltpu.sync_copy(acc.at[rep2d.at[j]],
                                rows_v.at[pl.ds(j * CH, CH)])
                pltpu.sync_copy(cnt.at[rep2d.at[j]],
                                ones1d.at[pl.ds(j * CH, CH)])
            pltpu.sync_copy(rows_v, S_o.at[pl.ds(base, EV)])
            pltpu.sync_copy(ones1d, c_o.at[pl.ds(base, EV)])

    out_type = (
        jax.ShapeDtypeStruct((B, D), jnp.float32),
        jax.ShapeDtypeStruct((B,), jnp.float32),
    )
    scratch = [
        pltpu.VMEM((EV // CH, CH), jnp.int32),
        pltpu.VMEM((EV,), jnp.float32),
        pltpu.VMEM((EV,), jnp.float32),
        pltpu.VMEM((EV, D), jnp.float32),
        pltpu.VMEM((EV, D), jnp.float32),
        pltpu.VMEM_SHARED((B, D), jnp.float32),
        pltpu.VMEM_SHARED((B,), jnp.float32),
    ]
    return pl.kernel(body, out_type=out_type, mesh=_sc_mesh(),
                     scratch_types=scratch, compiler_params=_sc_params())


def _make_gru(B, D, block):
    def body(S, c, h, Wr, Wz, Wn, Ur, Uz, Un, br, bz, bn, dr, dz, dn, out):
        m = S[...] / jnp.maximum(c[...], 1.0)
        hh = h[...]
        i_r = jnp.dot(m, Wr[...], preferred_element_type=jnp.float32) + br[...]
        i_z = jnp.dot(m, Wz[...], preferred_element_type=jnp.float32) + bz[...]
        i_n = jnp.dot(m, Wn[...], preferred_element_type=jnp.float32) + bn[...]
        h_r = jnp.dot(hh, Ur[...], preferred_element_type=jnp.float32) + dr[...]
        h_z = jnp.dot(hh, Uz[...], preferred_element_type=jnp.float32) + dz[...]
        h_n = jnp.dot(hh, Un[...], preferred_element_type=jnp.float32) + dn[...]
        r = jax.nn.sigmoid(i_r + h_r)
        z = jax.nn.sigmoid(i_z + h_z)
        n = jnp.tanh(i_n + r * h_n)
        out[:, pl.ds(0, D)] = ((1.0 - z) * n + z * hh)

    grid = (B // block,)
    row_spec = pl.BlockSpec((block, D), lambda i: (i, 0))
    c_spec = pl.BlockSpec((block, 1), lambda i: (i, 0))
    w_spec = pl.BlockSpec((D, D), lambda i: (0, 0))
    b_spec = pl.BlockSpec((1, D), lambda i: (0, 0))
    return pl.pallas_call(
        body,
        grid=grid,
        in_specs=[row_spec, c_spec, row_spec] + [w_spec] * 6 + [b_spec] * 6,
        out_specs=row_spec,
        out_shape=jax.ShapeDtypeStruct((B, D), jnp.float32),
    )


def _make_scatter(M, B, D):
    NW = NC * NS
    EV = B // NW
    NCHUNK = EV // CH

    def body(tab_in, idx2, hnew, out, idx2d, h_v):
        cid = lax.axis_index("c")
        sid = lax.axis_index("s")
        w = cid * NS + sid
        rows_per_w = EV // CH
        pltpu.sync_copy(idx2.at[pl.ds(w * rows_per_w, rows_per_w)], idx2d)
        _row_permute(idx2d, rows_per_w)
        pltpu.sync_copy(hnew.at[pl.ds(w * EV, EV)], h_v)
        for j in range(NCHUNK):
            pltpu.sync_copy(h_v.at[pl.ds(j * CH, CH)], out.at[idx2d.at[j]])

    scratch = [
        pltpu.VMEM((EV // CH, CH), jnp.int32),
        pltpu.VMEM((EV, D), jnp.float32),
    ]
    return _mpmd._mpmd_map(
        [(_sc_mesh(), body)],
        out_types=jax.ShapeDtypeStruct((M, D), jnp.float32),
        input_output_aliases={0: 0},
        scratch_types=scratch,
        compiler_params=_sc_params(),
    )


def kernel(mem, idx, val, W_ih, W_hh, b_ih, b_hh):
    M, D = mem.shape
    B = idx.shape[0]

    idx2 = idx.astype(jnp.int32).reshape(B // CH, CH)

    mem_t = jnp.swapaxes(mem, 0, 1)
    LB = 8192
    tab = _make_to_rows(M, D, LB)(mem_t)
    TL = tab.shape[0]
    tab1m = tab.reshape(TL * (W // D), D)

    rep2 = _make_elect(M, B)(idx2)
    S, c = _make_aggregate(B, D)(rep2, val)
    h = _make_hgather(M, B, D)(tab1m, idx2)

    Wr, Wz, Wn = W_ih[:, :D], W_ih[:, D:2 * D], W_ih[:, 2 * D:]
    Ur, Uz, Un = W_hh[:, :D], W_hh[:, D:2 * D], W_hh[:, 2 * D:]
    br, bz, bn = (b_ih[:D].reshape(1, D), b_ih[D:2 * D].reshape(1, D),
                  b_ih[2 * D:].reshape(1, D))
    dr, dz, dn = (b_hh[:D].reshape(1, D), b_hh[D:2 * D].reshape(1, D),
                  b_hh[2 * D:].reshape(1, D))

    h_new = _make_gru(B, D, 2048)(
        S, c.reshape(B, 1), h, Wr, Wz, Wn, Ur, Uz, Un,
        br, bz, bn, dr, dz, dn)

    out_tab = _make_scatter(tab1m.shape[0], B, D)(tab1m, idx2, h_new)
    out_t = _make_to_native(M, D, LB)(out_tab.reshape(TL, W))
    return jnp.swapaxes(out_t, 0, 1)
